# baseline (speedup 1.0000x reference)
import functools

import jax
import jax.numpy as jnp
from jax import lax
from jax.experimental import pallas as pl
from jax.experimental.pallas import tpu as pltpu
from jax.experimental.pallas import tpu_sc as plsc

D_MODEL = 128
BATCH = 4096
HIST = 50
B_TOTAL = BATCH * HIST

NUM_CORES = 2
NUM_SUBCORES = 16
NUM_WORKERS = NUM_CORES * NUM_SUBCORES
COLS_PER_W = BATCH // NUM_WORKERS

CHUNK = 128
N_CHUNKS = HIST
NBUF = 7
LAG = 4


def _make_gather():
    mesh = plsc.VectorSubcoreMesh(core_axis_name="c", subcore_axis_name="s")

    @functools.partial(
        pl.kernel,
        mesh=mesh,
        out_type=jax.ShapeDtypeStruct((B_TOTAL, D_MODEL), jnp.float32),
        compiler_params=pltpu.CompilerParams(use_tc_tiling_on_sc=True),
        scratch_types=[
            pltpu.VMEM((HIST, CHUNK), jnp.int32),
            pltpu.VMEM((NBUF * CHUNK, D_MODEL), jnp.float32),
            pltpu.SemaphoreType.DMA((NBUF,)),
            pltpu.SemaphoreType.DMA((NBUF,)),
        ],
    )
    def gather_kernel(idx_hbm, table_hbm, out_hbm, idx_v, rows_v, sem_g, sem_o):
        wid = lax.axis_index("s") * NUM_CORES + lax.axis_index("c")
        col0 = wid * COLS_PER_W

        def fire_gather(v, b):
            return pltpu.async_copy(
                table_hbm.at[idx_v.at[v]],
                rows_v.at[pl.ds(b * CHUNK, CHUNK)],
                sem_g.at[b],
            )

        def fire_out(v, b):
            return pltpu.async_copy(
                rows_v.at[pl.ds(b * CHUNK, CHUNK)],
                out_hbm.at[pl.ds(v * BATCH + col0, CHUNK)],
                sem_o.at[b],
            )

        def wait_gather(b):
            pltpu.make_async_copy(
                table_hbm.at[idx_v.at[0]],
                rows_v.at[pl.ds(b * CHUNK, CHUNK)],
                sem_g.at[b],
            ).wait()

        def wait_out(b):
            pltpu.make_async_copy(
                rows_v.at[pl.ds(b * CHUNK, CHUNK)],
                out_hbm.at[pl.ds(col0, CHUNK)],
                sem_o.at[b],
            ).wait()

        pltpu.sync_copy(
            idx_hbm.at[pl.ds(0, HIST), pl.ds(col0, CHUNK)], idx_v
        )

        def visit(v, _):
            b = lax.rem(v, NBUF)

            @pl.when(jnp.logical_and(v >= NBUF, v < N_CHUNKS))
            def _():
                wait_out(b)

            @pl.when(v < N_CHUNKS)
            def _():
                fire_gather(v, b)

            @pl.when(v >= LAG)
            def _():
                vl = v - LAG
                bl = lax.rem(vl, NBUF)
                wait_gather(bl)
                fire_out(vl, bl)

            return 0

        lax.fori_loop(0, N_CHUNKS + LAG, visit, 0)

        def drain(b, _):
            wait_out(b)
            return 0

        lax.fori_loop(0, NBUF, drain, 0)

    return gather_kernel


_gather = _make_gather()


def kernel(tokenized_precursor, table):
    idx = tokenized_precursor.astype(jnp.int32).T
    out = _gather(idx, table)
    return out.reshape(HIST, BATCH, D_MODEL).transpose(1, 0, 2)


# speedup vs baseline: 1.0127x; 1.0127x over previous
import functools

import jax
import jax.numpy as jnp
from jax import lax
from jax.experimental import pallas as pl
from jax.experimental.pallas import tpu as pltpu
from jax.experimental.pallas import tpu_sc as plsc

D_MODEL = 128
BATCH = 4096
HIST = 50
B_TOTAL = BATCH * HIST

NUM_CORES = 2
NUM_SUBCORES = 16
NUM_WORKERS = NUM_CORES * NUM_SUBCORES
COLS_PER_W = BATCH // NUM_WORKERS

CHUNK = 128
N_CHUNKS = HIST
NBUF = 6
LAG = 2


def _make_gather():
    mesh = plsc.VectorSubcoreMesh(core_axis_name="c", subcore_axis_name="s")

    @functools.partial(
        pl.kernel,
        mesh=mesh,
        out_type=jax.ShapeDtypeStruct((B_TOTAL, D_MODEL), jnp.float32),
        compiler_params=pltpu.CompilerParams(use_tc_tiling_on_sc=True),
        scratch_types=[
            pltpu.VMEM((HIST, CHUNK), jnp.int32),
            pltpu.VMEM((NBUF * CHUNK, D_MODEL), jnp.float32),
            pltpu.SemaphoreType.DMA((NBUF,)),
            pltpu.SemaphoreType.DMA((NBUF,)),
        ],
    )
    def gather_kernel(idx_hbm, table_hbm, out_hbm, idx_v, rows_v, sem_g, sem_o):
        wid = lax.axis_index("s") * NUM_CORES + lax.axis_index("c")
        col0 = wid * COLS_PER_W

        def fire_gather(v, b):
            return pltpu.async_copy(
                table_hbm.at[idx_v.at[v]],
                rows_v.at[pl.ds(b * CHUNK, CHUNK)],
                sem_g.at[b],
            )

        def fire_out(v, b):
            return pltpu.async_copy(
                rows_v.at[pl.ds(b * CHUNK, CHUNK)],
                out_hbm.at[pl.ds(v * BATCH + col0, CHUNK)],
                sem_o.at[b],
            )

        def wait_gather(b):
            pltpu.make_async_copy(
                table_hbm.at[idx_v.at[0]],
                rows_v.at[pl.ds(b * CHUNK, CHUNK)],
                sem_g.at[b],
            ).wait()

        def wait_out(b):
            pltpu.make_async_copy(
                rows_v.at[pl.ds(b * CHUNK, CHUNK)],
                out_hbm.at[pl.ds(col0, CHUNK)],
                sem_o.at[b],
            ).wait()

        pltpu.sync_copy(
            idx_hbm.at[pl.ds(0, HIST), pl.ds(col0, CHUNK)], idx_v
        )

        def visit(v, _):
            b = lax.rem(v, NBUF)

            @pl.when(jnp.logical_and(v >= NBUF, v < N_CHUNKS))
            def _():
                wait_out(b)

            @pl.when(v < N_CHUNKS)
            def _():
                fire_gather(v, b)

            @pl.when(v >= LAG)
            def _():
                vl = v - LAG
                bl = lax.rem(vl, NBUF)
                wait_gather(bl)
                fire_out(vl, bl)

            return 0

        lax.fori_loop(0, N_CHUNKS + LAG, visit, 0)

        def drain(b, _):
            wait_out(b)
            return 0

        lax.fori_loop(0, NBUF, drain, 0)

    return gather_kernel


_gather = _make_gather()


def kernel(tokenized_precursor, table):
    idx = tokenized_precursor.astype(jnp.int32).T
    out = _gather(idx, table)
    return out.reshape(HIST, BATCH, D_MODEL).transpose(1, 0, 2)


# speedup vs baseline: 1.0192x; 1.0065x over previous
import functools

import jax
import jax.numpy as jnp
from jax import lax
from jax.experimental import pallas as pl
from jax.experimental.pallas import tpu as pltpu
from jax.experimental.pallas import tpu_sc as plsc

D_MODEL = 128
BATCH = 4096
HIST = 50
B_TOTAL = BATCH * HIST

NUM_CORES = 2
NUM_SUBCORES = 16
NUM_WORKERS = NUM_CORES * NUM_SUBCORES
COLS_PER_W = BATCH // NUM_WORKERS

CHUNK = 128
N_CHUNKS = HIST
NBUF = 6
LAG = 3


def _make_gather():
    mesh = plsc.VectorSubcoreMesh(core_axis_name="c", subcore_axis_name="s")

    @functools.partial(
        pl.kernel,
        mesh=mesh,
        out_type=jax.ShapeDtypeStruct((B_TOTAL, D_MODEL), jnp.float32),
        compiler_params=pltpu.CompilerParams(use_tc_tiling_on_sc=True),
        scratch_types=[
            pltpu.VMEM((HIST, CHUNK), jnp.int32),
            pltpu.VMEM((NBUF * CHUNK, D_MODEL), jnp.float32),
            pltpu.SemaphoreType.DMA((NBUF,)),
            pltpu.SemaphoreType.DMA((NBUF,)),
        ],
    )
    def gather_kernel(idx_hbm, table_hbm, out_hbm, idx_v, rows_v, sem_g, sem_o):
        wid = lax.axis_index("s") * NUM_CORES + lax.axis_index("c")
        col0 = wid * COLS_PER_W

        def fire_gather(v, b):
            return pltpu.async_copy(
                table_hbm.at[idx_v.at[v]],
                rows_v.at[pl.ds(b * CHUNK, CHUNK)],
                sem_g.at[b],
            )

        def fire_out(v, b):
            return pltpu.async_copy(
                rows_v.at[pl.ds(b * CHUNK, CHUNK)],
                out_hbm.at[pl.ds(v * BATCH + col0, CHUNK)],
                sem_o.at[b],
            )

        def wait_gather(b):
            pltpu.make_async_copy(
                table_hbm.at[idx_v.at[0]],
                rows_v.at[pl.ds(b * CHUNK, CHUNK)],
                sem_g.at[b],
            ).wait()

        def wait_out(b):
            pltpu.make_async_copy(
                rows_v.at[pl.ds(b * CHUNK, CHUNK)],
                out_hbm.at[pl.ds(col0, CHUNK)],
                sem_o.at[b],
            ).wait()

        pltpu.sync_copy(
            idx_hbm.at[pl.ds(0, HIST), pl.ds(col0, CHUNK)], idx_v
        )

        def visit(v, _):
            b = lax.rem(v, NBUF)

            @pl.when(jnp.logical_and(v >= NBUF, v < N_CHUNKS))
            def _():
                wait_out(b)

            @pl.when(v < N_CHUNKS)
            def _():
                fire_gather(v, b)

            @pl.when(v >= LAG)
            def _():
                vl = v - LAG
                bl = lax.rem(vl, NBUF)
                wait_gather(bl)
                fire_out(vl, bl)

            return 0

        lax.fori_loop(0, N_CHUNKS + LAG, visit, 0)

        def drain(b, _):
            wait_out(b)
            return 0

        lax.fori_loop(0, NBUF, drain, 0)

    return gather_kernel


_gather = _make_gather()


def kernel(tokenized_precursor, table):
    idx = tokenized_precursor.astype(jnp.int32).T
    out = _gather(idx, table)
    return out.reshape(HIST, BATCH, D_MODEL).transpose(1, 0, 2)
